# Initial kernel scaffold; baseline (speedup 1.0000x reference)
import jax
import jax.numpy as jnp
from jax.experimental import pallas as pl


def kernel(x, W_in, W_out, W_gate, gate_w, up_w, down_w):
    raise NotImplementedError("write your pallas kernel here")


# speedup vs baseline: 1.0476x; 1.0476x over previous
import functools

import jax
import jax.numpy as jnp
from jax.experimental import pallas as pl
from jax.experimental.pallas import tpu as pltpu

H = 1024
FFN = 4096
E = 8
K = 2
T = 4096

BT = 512
BF = 1024


def _proj_router_body(x_ref, win_ref, wgate_ref, h_ref, wfull_ref):
    x = x_ref[...]
    h = jax.lax.dot_general(x, win_ref[...], (((1,), (1,)), ((), ())),
                            preferred_element_type=jnp.float32)
    h_ref[...] = h
    logits = jax.lax.dot_general(h, wgate_ref[...], (((1,), (1,)), ((), ())),
                                 preferred_element_type=jnp.float32)
    probs = jax.nn.softmax(logits, axis=-1)
    lane = jax.lax.broadcasted_iota(jnp.int32, probs.shape, 1)
    p1 = jnp.max(probs, axis=-1, keepdims=True)
    i1 = jnp.argmax(probs, axis=-1)
    masked = jnp.where(lane == i1[:, None], -jnp.inf, probs)
    p2 = jnp.max(masked, axis=-1, keepdims=True)
    i2 = jnp.argmax(masked, axis=-1)
    e2 = jnp.exp(p2 - p1)
    w1 = 1.0 / (1.0 + e2)
    w2 = e2 / (1.0 + e2)
    wfull = (jnp.where(lane == i1[:, None], w1, 0.0)
             + jnp.where(lane == i2[:, None], w2, 0.0))
    wfull_ref[...] = wfull


def _moe_body(h_ref, gate_ref, up_ref, down_ref, wfull_ref, out_ref):
    e = pl.program_id(1)
    f = pl.program_id(2)

    @pl.when(jnp.logical_and(e == 0, f == 0))
    def _():
        out_ref[...] = jnp.zeros_like(out_ref)

    h = h_ref[...]
    g = jax.lax.dot_general(h, gate_ref[0], (((1,), (1,)), ((), ())),
                            preferred_element_type=jnp.float32)
    u = jax.lax.dot_general(h, up_ref[0], (((1,), (1,)), ((), ())),
                            preferred_element_type=jnp.float32)
    p = jax.nn.silu(g) * u
    o = jax.lax.dot_general(p, down_ref[0], (((1,), (1,)), ((), ())),
                            preferred_element_type=jnp.float32)
    lane = jax.lax.broadcasted_iota(jnp.int32, wfull_ref.shape, 1)
    w_e = jnp.sum(wfull_ref[...] * (lane == e).astype(jnp.float32), axis=-1)
    out_ref[...] += o * w_e[:, None]


def _outproj_body(y_ref, wout_ref, out_ref):
    out_ref[...] = jax.lax.dot_general(
        y_ref[...], wout_ref[...], (((1,), (1,)), ((), ())),
        preferred_element_type=jnp.float32)


@jax.jit
def kernel(x, W_in, W_out, W_gate, gate_w, up_w, down_w):
    h, wfull = pl.pallas_call(
        _proj_router_body,
        grid=(T // BT,),
        in_specs=[
            pl.BlockSpec((BT, H), lambda t: (t, 0)),
            pl.BlockSpec((H, H), lambda t: (0, 0)),
            pl.BlockSpec((E, H), lambda t: (0, 0)),
        ],
        out_specs=[
            pl.BlockSpec((BT, H), lambda t: (t, 0)),
            pl.BlockSpec((BT, E), lambda t: (t, 0)),
        ],
        out_shape=[
            jax.ShapeDtypeStruct((T, H), jnp.float32),
            jax.ShapeDtypeStruct((T, E), jnp.float32),
        ],
    )(x, W_in, W_gate)

    y = pl.pallas_call(
        _moe_body,
        grid=(T // BT, E, FFN // BF),
        in_specs=[
            pl.BlockSpec((BT, H), lambda t, e, f: (t, 0)),
            pl.BlockSpec((1, BF, H), lambda t, e, f: (e, f, 0)),
            pl.BlockSpec((1, BF, H), lambda t, e, f: (e, f, 0)),
            pl.BlockSpec((1, H, BF), lambda t, e, f: (e, 0, f)),
            pl.BlockSpec((BT, E), lambda t, e, f: (t, 0)),
        ],
        out_specs=pl.BlockSpec((BT, H), lambda t, e, f: (t, 0)),
        out_shape=jax.ShapeDtypeStruct((T, H), jnp.float32),
    )(h, gate_w, up_w, down_w, wfull)

    out = pl.pallas_call(
        _outproj_body,
        grid=(T // BT,),
        in_specs=[
            pl.BlockSpec((BT, H), lambda t: (t, 0)),
            pl.BlockSpec((H, H), lambda t: (0, 0)),
        ],
        out_specs=pl.BlockSpec((BT, H), lambda t: (t, 0)),
        out_shape=jax.ShapeDtypeStruct((T, H), jnp.float32),
    )(y, W_out)
    return out


# speedup vs baseline: 2.1642x; 2.0658x over previous
import functools

import jax
import jax.numpy as jnp
from jax.experimental import pallas as pl
from jax.experimental.pallas import tpu as pltpu
from jax.experimental.pallas import tpu_sc as plsc

H = 1024
FFN = 4096
E = 8
K = 2
T = 4096

BT = 512
BROW = 512
BF = 1024
NPAD = T * K + E * BROW
G = NPAD // BROW
NF = FFN // BF
CH = 128
NCH = T // CH
SCW = 32


def _dotT(a, b):
    return jax.lax.dot_general(a, b, (((1,), (1,)), ((), ())),
                               preferred_element_type=jnp.float32)


def _router_body(x_ref, win_ref, wgate_ref, h_ref, logits_ref):
    h = _dotT(x_ref[...], win_ref[...])
    h_ref[...] = h
    logits_ref[...] = _dotT(h, wgate_ref[...])


def _index_body(logits_ref, pos0_ref, pos1_ref, w0_ref, w1_ref, te_ref,
                used_ref):
    logits = logits_ref[...]
    probs = jax.nn.softmax(logits, axis=-1)
    lane = jax.lax.broadcasted_iota(jnp.int32, probs.shape, 1)
    p1 = jnp.max(probs, axis=-1, keepdims=True)
    i1 = jnp.argmax(probs, axis=-1)
    oh1 = (lane == i1[:, None]).astype(jnp.float32)
    masked = jnp.where(oh1 > 0, -1.0, probs)
    p2 = jnp.max(masked, axis=-1, keepdims=True)
    i2 = jnp.argmax(masked, axis=-1)
    oh2 = (lane == i2[:, None]).astype(jnp.float32)
    e2 = jnp.exp(p2 - p1)
    w1v = 1.0 / (1.0 + e2)
    w2v = e2 / (1.0 + e2)
    w0_ref[...] = jnp.broadcast_to(w1v, (T, E))
    w1_ref[...] = jnp.broadcast_to(w2v, (T, E))

    oh = oh1 + oh2
    sub = jax.lax.broadcasted_iota(jnp.int32, (CH, CH), 0)
    lan2 = jax.lax.broadcasted_iota(jnp.int32, (CH, CH), 1)
    tri_incl = (lan2 <= sub).astype(jnp.float32)
    incl_chunks = []
    totals = []
    for c in range(NCH):
        blk = oh[c * CH:(c + 1) * CH, :]
        incl = jnp.dot(tri_incl, blk, preferred_element_type=jnp.float32)
        incl_chunks.append(incl)
        totals.append(incl[CH - 1:CH, :])
    tot = jnp.concatenate(totals, axis=0)
    sub3 = jax.lax.broadcasted_iota(jnp.int32, (NCH, NCH), 0)
    lan3 = jax.lax.broadcasted_iota(jnp.int32, (NCH, NCH), 1)
    tri_strict = (lan3 < sub3).astype(jnp.float32)
    excl_tot = jnp.dot(tri_strict, tot, preferred_element_type=jnp.float32)
    incl_all = jnp.concatenate(
        [incl_chunks[c] + excl_tot[c:c + 1, :] for c in range(NCH)], axis=0)
    counts = jnp.sum(tot, axis=0, keepdims=True)
    padded = jnp.floor((counts + (BROW - 1)) / BROW) * BROW
    sub4 = jax.lax.broadcasted_iota(jnp.int32, (E, E), 0)
    lan4 = jax.lax.broadcasted_iota(jnp.int32, (E, E), 1)
    lt = (sub4 < lan4).astype(jnp.float32)
    offsets = jnp.dot(padded, lt, preferred_element_type=jnp.float32)
    excl = incl_all - oh
    pos_e = offsets + excl
    pos0 = jnp.sum(oh1 * pos_e, axis=-1, keepdims=True)
    pos1 = jnp.sum(oh2 * pos_e, axis=-1, keepdims=True)
    pos0_ref[...] = jnp.broadcast_to(pos0, (T, E)).astype(jnp.int32)
    pos1_ref[...] = jnp.broadcast_to(pos1, (T, E)).astype(jnp.int32)
    total_used = jnp.sum(padded, axis=-1, keepdims=True)
    used_ref[...] = jnp.broadcast_to(
        total_used / BROW, (1, E)).astype(jnp.int32)
    g_iota = (jax.lax.broadcasted_iota(jnp.int32, (1, G), 1) * BROW
              ).astype(jnp.float32)
    te = jnp.zeros((1, G), jnp.float32)
    for e in range(E):
        te = te + (g_iota >= offsets[0, e]).astype(jnp.float32)
    te_ref[...] = (te - 1.0).astype(jnp.int32)


NSHARD = 32
NTOK = T // NSHARD
NSUB = NTOK // SCW


def _dispatch(h, pos0, pos1):
    mesh = plsc.VectorSubcoreMesh(core_axis_name="c", subcore_axis_name="s")

    @functools.partial(
        pl.kernel,
        out_type=jax.ShapeDtypeStruct((NPAD, H), jnp.float32),
        mesh=mesh,
        scratch_types=[pltpu.VMEM((1, NTOK), jnp.int32),
                       pltpu.VMEM((1, NTOK), jnp.int32),
                       pltpu.VMEM((SCW, H), jnp.float32),
                       pltpu.SemaphoreType.DMA,
                       pltpu.SemaphoreType.DMA])
    def run(h_hbm, p0_hbm, p1_hbm, hg_hbm, i0b, i1b, rowb, sem0, sem1):
        c = jax.lax.axis_index("c")
        s = jax.lax.axis_index("s")
        base = (c * 16 + s) * NTOK
        pltpu.async_copy(p0_hbm.at[:, pl.ds(base, NTOK)], i0b, sem0).wait()
        pltpu.async_copy(p1_hbm.at[:, pl.ds(base, NTOK)], i1b, sem1).wait()

        @pl.loop(0, NSUB)
        def _(sub):
            r0 = base + sub * SCW
            pltpu.async_copy(h_hbm.at[pl.ds(r0, SCW), :], rowb, sem0).wait()
            cp0 = pltpu.async_copy(
                rowb, hg_hbm.at[i0b.at[0, pl.ds(sub * SCW, SCW)]], sem0)
            cp1 = pltpu.async_copy(
                rowb, hg_hbm.at[i1b.at[0, pl.ds(sub * SCW, SCW)]], sem1)
            cp0.wait()
            cp1.wait()

    return run(h, pos0, pos1)


def _combine_gather(og, pos0, pos1):
    mesh = plsc.VectorSubcoreMesh(core_axis_name="c", subcore_axis_name="s")

    @functools.partial(
        pl.kernel,
        out_type=[jax.ShapeDtypeStruct((T, H), jnp.float32),
                  jax.ShapeDtypeStruct((T, H), jnp.float32)],
        mesh=mesh,
        scratch_types=[pltpu.VMEM((1, NTOK), jnp.int32),
                       pltpu.VMEM((1, NTOK), jnp.int32),
                       pltpu.VMEM((SCW, H), jnp.float32),
                       pltpu.VMEM((SCW, H), jnp.float32),
                       pltpu.SemaphoreType.DMA,
                       pltpu.SemaphoreType.DMA])
    def run(og_hbm, p0_hbm, p1_hbm, o0_hbm, o1_hbm, i0b, i1b, rb0, rb1,
            sem0, sem1):
        c = jax.lax.axis_index("c")
        s = jax.lax.axis_index("s")
        base = (c * 16 + s) * NTOK
        pltpu.async_copy(p0_hbm.at[:, pl.ds(base, NTOK)], i0b, sem0).wait()
        pltpu.async_copy(p1_hbm.at[:, pl.ds(base, NTOK)], i1b, sem1).wait()

        @pl.loop(0, NSUB)
        def _(sub):
            r0 = base + sub * SCW
            cp0 = pltpu.async_copy(
                og_hbm.at[i0b.at[0, pl.ds(sub * SCW, SCW)]], rb0, sem0)
            cp1 = pltpu.async_copy(
                og_hbm.at[i1b.at[0, pl.ds(sub * SCW, SCW)]], rb1, sem1)
            cp0.wait()
            cp1.wait()
            cp0 = pltpu.async_copy(rb0, o0_hbm.at[pl.ds(r0, SCW), :], sem0)
            cp1 = pltpu.async_copy(rb1, o1_hbm.at[pl.ds(r0, SCW), :], sem1)
            cp0.wait()
            cp1.wait()

    return run(og, pos0, pos1)


def _grouped_body(te_ref, used_ref, hg_ref, gate_ref, up_ref, down_ref,
                  og_ref):
    g = pl.program_id(0)
    f = pl.program_id(1)

    @pl.when(g < used_ref[0])
    def _():
        hg = hg_ref[...]
        gv = _dotT(hg, gate_ref[0])
        uv = _dotT(hg, up_ref[0])
        p = jax.nn.silu(gv) * uv
        o = jax.lax.dot_general(p, down_ref[0], (((1,), (1,)), ((), ())),
                                preferred_element_type=jnp.float32)

        @pl.when(f == 0)
        def _():
            og_ref[...] = o

        @pl.when(f > 0)
        def _():
            og_ref[...] += o


def _final_body(o0_ref, o1_ref, w0_ref, w1_ref, wout_ref, out_ref):
    y = (o0_ref[...] * w0_ref[:, 0:1] + o1_ref[...] * w1_ref[:, 0:1])
    out_ref[...] = _dotT(y, wout_ref[...])


@jax.jit
def kernel(x, W_in, W_out, W_gate, gate_w, up_w, down_w):
    h, logits = pl.pallas_call(
        _router_body,
        grid=(T // BT,),
        in_specs=[
            pl.BlockSpec((BT, H), lambda t: (t, 0)),
            pl.BlockSpec((H, H), lambda t: (0, 0)),
            pl.BlockSpec((E, H), lambda t: (0, 0)),
        ],
        out_specs=[
            pl.BlockSpec((BT, H), lambda t: (t, 0)),
            pl.BlockSpec((BT, E), lambda t: (t, 0)),
        ],
        out_shape=[
            jax.ShapeDtypeStruct((T, H), jnp.float32),
            jax.ShapeDtypeStruct((T, E), jnp.float32),
        ],
    )(x, W_in, W_gate)

    pos0b, pos1b, w0b, w1b, te2, used2 = pl.pallas_call(
        _index_body,
        grid=(1,),
        in_specs=[pl.BlockSpec((T, E), lambda i: (0, 0))],
        out_specs=[
            pl.BlockSpec((T, E), lambda i: (0, 0)),
            pl.BlockSpec((T, E), lambda i: (0, 0)),
            pl.BlockSpec((T, E), lambda i: (0, 0)),
            pl.BlockSpec((T, E), lambda i: (0, 0)),
            pl.BlockSpec((1, G), lambda i: (0, 0)),
            pl.BlockSpec((1, E), lambda i: (0, 0)),
        ],
        out_shape=[
            jax.ShapeDtypeStruct((T, E), jnp.int32),
            jax.ShapeDtypeStruct((T, E), jnp.int32),
            jax.ShapeDtypeStruct((T, E), jnp.float32),
            jax.ShapeDtypeStruct((T, E), jnp.float32),
            jax.ShapeDtypeStruct((1, G), jnp.int32),
            jax.ShapeDtypeStruct((1, E), jnp.int32),
        ],
    )(logits)

    pos0 = pos0b[:, 0].reshape(1, T)
    pos1 = pos1b[:, 0].reshape(1, T)
    te = te2.reshape(G)
    used = used2[0, 0:1]

    hg = _dispatch(h, pos0, pos1)

    og = pl.pallas_call(
        _grouped_body,
        grid_spec=pltpu.PrefetchScalarGridSpec(
            num_scalar_prefetch=2,
            grid=(G, NF),
            in_specs=[
                pl.BlockSpec((BROW, H), lambda g, f, te_r, u_r: (g, 0)),
                pl.BlockSpec((1, BF, H),
                             lambda g, f, te_r, u_r: (te_r[g], f, 0)),
                pl.BlockSpec((1, BF, H),
                             lambda g, f, te_r, u_r: (te_r[g], f, 0)),
                pl.BlockSpec((1, H, BF),
                             lambda g, f, te_r, u_r: (te_r[g], 0, f)),
            ],
            out_specs=pl.BlockSpec((BROW, H), lambda g, f, te_r, u_r: (g, 0)),
        ),
        out_shape=jax.ShapeDtypeStruct((NPAD, H), jnp.float32),
    )(te, used, hg, gate_w, up_w, down_w)

    o0, o1 = _combine_gather(og, pos0, pos1)

    out = pl.pallas_call(
        _final_body,
        grid=(T // BT,),
        in_specs=[
            pl.BlockSpec((BT, H), lambda t: (t, 0)),
            pl.BlockSpec((BT, H), lambda t: (t, 0)),
            pl.BlockSpec((BT, E), lambda t: (t, 0)),
            pl.BlockSpec((BT, E), lambda t: (t, 0)),
            pl.BlockSpec((H, H), lambda t: (0, 0)),
        ],
        out_specs=pl.BlockSpec((BT, H), lambda t: (t, 0)),
        out_shape=jax.ShapeDtypeStruct((T, H), jnp.float32),
    )(o0, o1, w0b, w1b, W_out)
    return out


# speedup vs baseline: 2.3464x; 1.0842x over previous
import functools

import jax
import jax.numpy as jnp
from jax.experimental import pallas as pl
from jax.experimental.pallas import tpu as pltpu
from jax.experimental.pallas import tpu_sc as plsc

H = 1024
FFN = 4096
E = 8
K = 2
T = 4096

BT = 512
BROW = 512
BF = 1024
NPAD = T * K + E * BROW
G = NPAD // BROW
NF = FFN // BF
CH = 128
NCH = T // CH
SCW = 32


def _dotT(a, b):
    return jax.lax.dot_general(a, b, (((1,), (1,)), ((), ())),
                               preferred_element_type=jnp.float32)


def _router_body(x_ref, win_ref, wgate_ref, h_ref, logits_ref):
    h = _dotT(x_ref[...], win_ref[...])
    h_ref[...] = h
    logits_ref[...] = _dotT(h, wgate_ref[...])


def _index_body(logits_ref, pos0_ref, pos1_ref, w0_ref, w1_ref, te_ref,
                used_ref):
    logits = logits_ref[...]
    probs = jax.nn.softmax(logits, axis=-1)
    lane = jax.lax.broadcasted_iota(jnp.int32, probs.shape, 1)
    p1 = jnp.max(probs, axis=-1, keepdims=True)
    i1 = jnp.argmax(probs, axis=-1)
    oh1 = (lane == i1[:, None]).astype(jnp.float32)
    masked = jnp.where(oh1 > 0, -1.0, probs)
    p2 = jnp.max(masked, axis=-1, keepdims=True)
    i2 = jnp.argmax(masked, axis=-1)
    oh2 = (lane == i2[:, None]).astype(jnp.float32)
    e2 = jnp.exp(p2 - p1)
    w1v = 1.0 / (1.0 + e2)
    w2v = e2 / (1.0 + e2)
    w0_ref[...] = jnp.broadcast_to(w1v, (T, E))
    w1_ref[...] = jnp.broadcast_to(w2v, (T, E))

    oh = oh1 + oh2
    sub = jax.lax.broadcasted_iota(jnp.int32, (CH, CH), 0)
    lan2 = jax.lax.broadcasted_iota(jnp.int32, (CH, CH), 1)
    tri_incl = (lan2 <= sub).astype(jnp.float32)
    incl_chunks = []
    totals = []
    for c in range(NCH):
        blk = oh[c * CH:(c + 1) * CH, :]
        incl = jnp.dot(tri_incl, blk, preferred_element_type=jnp.float32)
        incl_chunks.append(incl)
        totals.append(incl[CH - 1:CH, :])
    tot = jnp.concatenate(totals, axis=0)
    sub3 = jax.lax.broadcasted_iota(jnp.int32, (NCH, NCH), 0)
    lan3 = jax.lax.broadcasted_iota(jnp.int32, (NCH, NCH), 1)
    tri_strict = (lan3 < sub3).astype(jnp.float32)
    excl_tot = jnp.dot(tri_strict, tot, preferred_element_type=jnp.float32)
    incl_all = jnp.concatenate(
        [incl_chunks[c] + excl_tot[c:c + 1, :] for c in range(NCH)], axis=0)
    counts = jnp.sum(tot, axis=0, keepdims=True)
    padded = jnp.floor((counts + (BROW - 1)) / BROW) * BROW
    sub4 = jax.lax.broadcasted_iota(jnp.int32, (E, E), 0)
    lan4 = jax.lax.broadcasted_iota(jnp.int32, (E, E), 1)
    lt = (sub4 < lan4).astype(jnp.float32)
    offsets = jnp.dot(padded, lt, preferred_element_type=jnp.float32)
    excl = incl_all - oh
    pos_e = offsets + excl
    pos0 = jnp.sum(oh1 * pos_e, axis=-1, keepdims=True)
    pos1 = jnp.sum(oh2 * pos_e, axis=-1, keepdims=True)
    pos0_ref[...] = jnp.broadcast_to(pos0, (T, E)).astype(jnp.int32)
    pos1_ref[...] = jnp.broadcast_to(pos1, (T, E)).astype(jnp.int32)
    total_used = jnp.sum(padded, axis=-1, keepdims=True)
    used_ref[...] = jnp.broadcast_to(
        total_used / BROW, (1, E)).astype(jnp.int32)
    g_iota = (jax.lax.broadcasted_iota(jnp.int32, (1, G), 1) * BROW
              ).astype(jnp.float32)
    te = jnp.zeros((1, G), jnp.float32)
    for e in range(E):
        te = te + (g_iota >= offsets[0, e]).astype(jnp.float32)
    te_ref[...] = (te - 1.0).astype(jnp.int32)


NSHARD = 32
NTOK = T // NSHARD
NSUB = NTOK // SCW


def _dispatch(h, pos0, pos1):
    mesh = plsc.VectorSubcoreMesh(core_axis_name="c", subcore_axis_name="s")

    @functools.partial(
        pl.kernel,
        out_type=jax.ShapeDtypeStruct((NPAD, H), jnp.float32),
        mesh=mesh,
        scratch_types=[pltpu.VMEM((1, NTOK), jnp.int32),
                       pltpu.VMEM((1, NTOK), jnp.int32),
                       pltpu.VMEM((SCW, H), jnp.float32),
                       pltpu.SemaphoreType.DMA,
                       pltpu.SemaphoreType.DMA])
    def run(h_hbm, p0_hbm, p1_hbm, hg_hbm, i0b, i1b, rowb, sem0, sem1):
        c = jax.lax.axis_index("c")
        s = jax.lax.axis_index("s")
        base = (c * 16 + s) * NTOK
        pltpu.async_copy(p0_hbm.at[:, pl.ds(base, NTOK)], i0b, sem0).wait()
        pltpu.async_copy(p1_hbm.at[:, pl.ds(base, NTOK)], i1b, sem1).wait()

        @pl.loop(0, NSUB)
        def _(sub):
            r0 = base + sub * SCW
            pltpu.async_copy(h_hbm.at[pl.ds(r0, SCW), :], rowb, sem0).wait()
            cp0 = pltpu.async_copy(
                rowb, hg_hbm.at[i0b.at[0, pl.ds(sub * SCW, SCW)]], sem0)
            cp1 = pltpu.async_copy(
                rowb, hg_hbm.at[i1b.at[0, pl.ds(sub * SCW, SCW)]], sem1)
            cp0.wait()
            cp1.wait()

    return run(h, pos0, pos1)


def _combine_gather(og, pos0, pos1):
    mesh = plsc.VectorSubcoreMesh(core_axis_name="c", subcore_axis_name="s")

    @functools.partial(
        pl.kernel,
        out_type=[jax.ShapeDtypeStruct((T, H), jnp.float32),
                  jax.ShapeDtypeStruct((T, H), jnp.float32)],
        mesh=mesh,
        scratch_types=[pltpu.VMEM((1, NTOK), jnp.int32),
                       pltpu.VMEM((1, NTOK), jnp.int32),
                       pltpu.VMEM((SCW, H), jnp.float32),
                       pltpu.VMEM((SCW, H), jnp.float32),
                       pltpu.SemaphoreType.DMA,
                       pltpu.SemaphoreType.DMA])
    def run(og_hbm, p0_hbm, p1_hbm, o0_hbm, o1_hbm, i0b, i1b, rb0, rb1,
            sem0, sem1):
        c = jax.lax.axis_index("c")
        s = jax.lax.axis_index("s")
        base = (c * 16 + s) * NTOK
        pltpu.async_copy(p0_hbm.at[:, pl.ds(base, NTOK)], i0b, sem0).wait()
        pltpu.async_copy(p1_hbm.at[:, pl.ds(base, NTOK)], i1b, sem1).wait()

        @pl.loop(0, NSUB)
        def _(sub):
            r0 = base + sub * SCW
            cp0 = pltpu.async_copy(
                og_hbm.at[i0b.at[0, pl.ds(sub * SCW, SCW)]], rb0, sem0)
            cp1 = pltpu.async_copy(
                og_hbm.at[i1b.at[0, pl.ds(sub * SCW, SCW)]], rb1, sem1)
            cp0.wait()
            cp1.wait()
            cp0 = pltpu.async_copy(rb0, o0_hbm.at[pl.ds(r0, SCW), :], sem0)
            cp1 = pltpu.async_copy(rb1, o1_hbm.at[pl.ds(r0, SCW), :], sem1)
            cp0.wait()
            cp1.wait()

    return run(og, pos0, pos1)


def _grouped_body(te_ref, used_ref, hg_ref, gate_ref, up_ref, down_ref,
                  og_ref):
    g = pl.program_id(0)
    f = pl.program_id(1)

    @pl.when(g < used_ref[0])
    def _():
        hg = hg_ref[...]
        gv = _dotT(hg, gate_ref[0])
        uv = _dotT(hg, up_ref[0])
        p = jax.nn.silu(gv) * uv
        o = jax.lax.dot_general(p, down_ref[0], (((1,), (1,)), ((), ())),
                                preferred_element_type=jnp.float32)

        @pl.when(f == 0)
        def _():
            og_ref[...] = o

        @pl.when(f > 0)
        def _():
            og_ref[...] += o


def _final_body(o0_ref, o1_ref, w0_ref, w1_ref, wout_ref, out_ref):
    y = (o0_ref[...] * w0_ref[:, 0:1] + o1_ref[...] * w1_ref[:, 0:1])
    out_ref[...] = _dotT(y, wout_ref[...])


@jax.jit
def kernel(x, W_in, W_out, W_gate, gate_w, up_w, down_w):
    h, logits = pl.pallas_call(
        _router_body,
        grid=(T // BT,),
        in_specs=[
            pl.BlockSpec((BT, H), lambda t: (t, 0)),
            pl.BlockSpec((H, H), lambda t: (0, 0)),
            pl.BlockSpec((E, H), lambda t: (0, 0)),
        ],
        out_specs=[
            pl.BlockSpec((BT, H), lambda t: (t, 0)),
            pl.BlockSpec((BT, E), lambda t: (t, 0)),
        ],
        out_shape=[
            jax.ShapeDtypeStruct((T, H), jnp.float32),
            jax.ShapeDtypeStruct((T, E), jnp.float32),
        ],
    )(x, W_in, W_gate)

    pos0b, pos1b, w0b, w1b, te2, used2 = pl.pallas_call(
        _index_body,
        grid=(1,),
        in_specs=[pl.BlockSpec((T, E), lambda i: (0, 0))],
        out_specs=[
            pl.BlockSpec((T, E), lambda i: (0, 0)),
            pl.BlockSpec((T, E), lambda i: (0, 0)),
            pl.BlockSpec((T, E), lambda i: (0, 0)),
            pl.BlockSpec((T, E), lambda i: (0, 0)),
            pl.BlockSpec((1, G), lambda i: (0, 0)),
            pl.BlockSpec((1, E), lambda i: (0, 0)),
        ],
        out_shape=[
            jax.ShapeDtypeStruct((T, E), jnp.int32),
            jax.ShapeDtypeStruct((T, E), jnp.int32),
            jax.ShapeDtypeStruct((T, E), jnp.float32),
            jax.ShapeDtypeStruct((T, E), jnp.float32),
            jax.ShapeDtypeStruct((1, G), jnp.int32),
            jax.ShapeDtypeStruct((1, E), jnp.int32),
        ],
    )(logits)

    pos0 = pos0b[:, 0].reshape(1, T)
    pos1 = pos1b[:, 0].reshape(1, T)
    te = te2.reshape(G)
    used = used2[0, 0:1]

    hg = _dispatch(h, pos0, pos1)

    og = pl.pallas_call(
        _grouped_body,
        grid_spec=pltpu.PrefetchScalarGridSpec(
            num_scalar_prefetch=2,
            grid=(G, NF),
            in_specs=[
                pl.BlockSpec(
                    (BROW, H),
                    lambda g, f, te_r, u_r: (jnp.minimum(g, u_r[0] - 1), 0)),
                pl.BlockSpec(
                    (1, BF, H),
                    lambda g, f, te_r, u_r:
                    (te_r[g], jnp.where(g < u_r[0], f, NF - 1), 0)),
                pl.BlockSpec(
                    (1, BF, H),
                    lambda g, f, te_r, u_r:
                    (te_r[g], jnp.where(g < u_r[0], f, NF - 1), 0)),
                pl.BlockSpec(
                    (1, H, BF),
                    lambda g, f, te_r, u_r:
                    (te_r[g], 0, jnp.where(g < u_r[0], f, NF - 1))),
            ],
            out_specs=pl.BlockSpec(
                (BROW, H),
                lambda g, f, te_r, u_r: (jnp.minimum(g, u_r[0] - 1), 0)),
        ),
        out_shape=jax.ShapeDtypeStruct((NPAD, H), jnp.float32),
    )(te, used, hg, gate_w, up_w, down_w)

    o0, o1 = _combine_gather(og, pos0, pos1)

    out = pl.pallas_call(
        _final_body,
        grid=(T // BT,),
        in_specs=[
            pl.BlockSpec((BT, H), lambda t: (t, 0)),
            pl.BlockSpec((BT, H), lambda t: (t, 0)),
            pl.BlockSpec((BT, E), lambda t: (t, 0)),
            pl.BlockSpec((BT, E), lambda t: (t, 0)),
            pl.BlockSpec((H, H), lambda t: (0, 0)),
        ],
        out_specs=pl.BlockSpec((BT, H), lambda t: (t, 0)),
        out_shape=jax.ShapeDtypeStruct((T, H), jnp.float32),
    )(o0, o1, w0b, w1b, W_out)
    return out
